# baseline (speedup 1.0000x reference)
import functools

import jax
import jax.numpy as jnp
from jax import lax
from jax.experimental import pallas as pl
from jax.experimental.pallas import tpu as pltpu
from jax.experimental.pallas import tpu_sc as plsc

_NC = 2
_NS = 16
_NW = _NC * _NS
_CHUNK = 128
_LANES = 16
_PACK = 4


def _gather_body(n_blocks, seq, bsz, emb_dim, idx_hbm, table_hbm, out_hbm,
                 idx_v, idxs_v, rows_v, tr_v, isem, gsems, osems):
  wid = lax.axis_index("s") * _NC + lax.axis_index("c")
  nb_per_l = bsz // _CHUNK
  b0 = wid * n_blocks
  srow = _PACK * emb_dim

  def blk(j):
    bid = b0 + j
    return bid // nb_per_l, bid % nb_per_l

  def stage(j, carry):
    l, c = blk(j)
    pltpu.make_async_copy(
        idx_hbm.at[l, pl.ds(c * _CHUNK, _CHUNK)], idx_v.at[j], isem).start()
    return carry

  lax.fori_loop(0, n_blocks, stage, 0, unroll=False)

  def drain_idx(j, carry):
    pltpu.make_async_copy(
        idx_hbm.at[0, pl.ds(0, _CHUNK)], idx_v.at[j], isem).wait()
    return carry

  lax.fori_loop(0, n_blocks, drain_idx, 0, unroll=False)

  n_grp = _CHUNK // _LANES
  true_mask = jnp.ones((_LANES,), jnp.bool_)
  iota = lax.iota(jnp.int32, _LANES)
  t_ids = [iota + g * _LANES for g in range(n_grp)]

  def build_super_idx(s, j):
    for g in range(n_grp):
      v = idx_v[j, pl.ds(g * _LANES, _LANES)]
      plsc.store_compressed(
          idxs_v.at[s, pl.ds(g * _LANES, _LANES)],
          jnp.right_shift(v, 2), mask=true_mask)

  def gather_copy(s, j):
    return pltpu.make_async_copy(table_hbm.at[idxs_v.at[s]], rows_v.at[s],
                                 gsems[s])

  def extract(s, j):
    rows_s = rows_v.at[s]
    for g in range(n_grp):
      v = idx_v[j, pl.ds(g * _LANES, _LANES)]
      col0 = jnp.bitwise_and(v, _PACK - 1) * emb_dim
      vecs = [plsc.load_gather(rows_s, [t_ids[g], col0 + d])
              for d in range(emb_dim)]
      for d in range(emb_dim):
        plsc.store_compressed(
            tr_v.at[s, d, pl.ds(g * _LANES, _LANES)], vecs[d],
            mask=true_mask)

  def out_copy(s, j):
    l, c = blk(j)
    return pltpu.make_async_copy(
        tr_v.at[s], out_hbm.at[l, :, pl.ds(c * _CHUNK, _CHUNK)], osems[s])

  build_super_idx(0, 0)
  gather_copy(0, 0).start()
  build_super_idx(1, 1)
  gather_copy(1, 1).start()

  n_pairs = n_blocks // 2 - 1

  def body(p, carry):
    j = 2 * p
    gather_copy(0, j).wait()
    extract(0, j)
    out_copy(0, j).start()
    gather_copy(1, j + 1).wait()
    extract(1, j + 1)
    out_copy(1, j + 1).start()
    out_copy(0, j).wait()
    build_super_idx(0, j + 2)
    gather_copy(0, j + 2).start()
    out_copy(1, j + 1).wait()
    build_super_idx(1, j + 3)
    gather_copy(1, j + 3).start()
    return carry

  lax.fori_loop(0, n_pairs, body, 0, unroll=False)

  j = 2 * n_pairs
  gather_copy(0, j).wait()
  extract(0, j)
  out_copy(0, j).start()
  gather_copy(1, j + 1).wait()
  extract(1, j + 1)
  out_copy(1, j + 1).start()
  out_copy(0, j).wait()
  out_copy(1, j + 1).wait()


@functools.partial(jax.jit, static_argnames=("seq", "bsz", "emb_dim"))
def _sc_gather(idx, table, *, seq, bsz, emb_dim):
  mesh = plsc.VectorSubcoreMesh(
      core_axis_name="c", subcore_axis_name="s",
      num_cores=_NC, num_subcores=_NS)
  n_blocks = seq * bsz // (_NW * _CHUNK)
  srow = _PACK * emb_dim
  run = pl.kernel(
      functools.partial(_gather_body, n_blocks, seq, bsz, emb_dim),
      out_type=jax.ShapeDtypeStruct((seq, emb_dim, bsz), jnp.float32),
      mesh=mesh,
      scratch_types=[
          pltpu.VMEM((n_blocks, _CHUNK), jnp.int32),
          pltpu.VMEM((2, _CHUNK), jnp.int32),
          pltpu.VMEM((2, _CHUNK, srow), jnp.float32),
          pltpu.VMEM((2, emb_dim, _CHUNK), jnp.float32),
          pltpu.SemaphoreType.DMA,
          [pltpu.SemaphoreType.DMA] * 2,
          [pltpu.SemaphoreType.DMA] * 2,
      ],
      compiler_params=pltpu.CompilerParams(use_tc_tiling_on_sc=False,
                                           needs_layout_passes=False),
  )
  return run(idx, table)


def kernel(indices, table):
  bsz, seq = indices.shape
  vocab, emb_dim = table.shape
  idx_t = indices.T.astype(jnp.int32)
  tab_s = table.reshape(vocab // _PACK, _PACK * emb_dim)
  out_t = _sc_gather(idx_t, tab_s, seq=seq, bsz=bsz, emb_dim=emb_dim)
  return out_t.transpose(2, 0, 1)


# speedup vs baseline: 1.1044x; 1.1044x over previous
import functools

import jax
import jax.numpy as jnp
from jax import lax
from jax.experimental import pallas as pl
from jax.experimental.pallas import tpu as pltpu
from jax.experimental.pallas import tpu_sc as plsc

_NC = 2
_NS = 16
_NW = _NC * _NS
_CHUNK = 128
_NBUF = 5


def _gather_body(n_blocks, seq, bsz, emb_dim, idx_hbm, table_hbm, out_hbm,
                 idx_v, rows_v, isem, gsems, osems):
  wid = lax.axis_index("s") * _NC + lax.axis_index("c")
  nb_per_l = bsz // _CHUNK
  b0 = wid * n_blocks

  def stage(j, carry):
    bid = b0 + j
    l = bid // nb_per_l
    c = bid % nb_per_l
    pltpu.make_async_copy(
        idx_hbm.at[l, pl.ds(c * _CHUNK, _CHUNK)], idx_v.at[j], isem).start()
    return carry

  lax.fori_loop(0, n_blocks, stage, 0, unroll=False)

  def drain(j, carry):
    pltpu.make_async_copy(
        idx_hbm.at[0, pl.ds(0, _CHUNK)], idx_v.at[j], isem).wait()
    return carry

  lax.fori_loop(0, n_blocks, drain, 0, unroll=False)

  def gather_copy(b, j):
    return pltpu.make_async_copy(table_hbm.at[idx_v.at[j]], rows_v.at[b],
                                 gsems[b])

  def out_copy(b, j):
    bid = b0 + j
    l = bid // nb_per_l
    c = bid % nb_per_l
    return pltpu.make_async_copy(
        rows_v.at[b], out_hbm.at[l, pl.ds(c * _CHUNK, _CHUNK)], osems[b])

  for b in range(_NBUF):
    gather_copy(b, b).start()

  n_steady = n_blocks // _NBUF - 1

  def body(i, carry):
    k = i * _NBUF
    for b in range(_NBUF):
      gather_copy(b, k + b).wait()
      out_copy(b, k + b).start()
    for b in range(_NBUF):
      out_copy(b, k + b).wait()
      gather_copy(b, k + b + _NBUF).start()
    return carry

  lax.fori_loop(0, n_steady, body, 0, unroll=False)

  k = n_steady * _NBUF
  for b in range(_NBUF):
    gather_copy(b, k + b).wait()
    out_copy(b, k + b).start()
  for b in range(_NBUF):
    out_copy(b, k + b).wait()


@functools.partial(jax.jit, static_argnames=("seq", "bsz", "emb_dim"))
def _sc_gather(idx, table, *, seq, bsz, emb_dim):
  mesh = plsc.VectorSubcoreMesh(
      core_axis_name="c", subcore_axis_name="s",
      num_cores=_NC, num_subcores=_NS)
  n_blocks = seq * bsz // (_NW * _CHUNK)
  run = pl.kernel(
      functools.partial(_gather_body, n_blocks, seq, bsz, emb_dim),
      out_type=jax.ShapeDtypeStruct((seq, bsz, emb_dim), jnp.float32),
      mesh=mesh,
      scratch_types=[
          pltpu.VMEM((n_blocks, _CHUNK), jnp.int32),
          pltpu.VMEM((_NBUF, _CHUNK, emb_dim), jnp.float32),
          pltpu.SemaphoreType.DMA,
          [pltpu.SemaphoreType.DMA] * _NBUF,
          [pltpu.SemaphoreType.DMA] * _NBUF,
      ],
      compiler_params=pltpu.CompilerParams(use_tc_tiling_on_sc=False),
  )
  return run(idx, table)


def kernel(indices, table):
  bsz, seq = indices.shape
  vocab, emb_dim = table.shape
  idx_t = indices.T.astype(jnp.int32)
  out_t = _sc_gather(idx_t, table, seq=seq, bsz=bsz, emb_dim=emb_dim)
  return out_t.transpose(1, 0, 2)
